# baseline (speedup 1.0000x reference)
import jax
import jax.numpy as jnp
from jax.experimental import pallas as pl
from jax.experimental.pallas import tpu as pltpu

GAMMA = 1.0
ALPHA = 0.5
BETA = 10.0
THETA = 1.0

B, N, M = 4, 16384, 1024
NB = 2048
NSTEPS = N // NB


def _body(sample_ref, xyzt_ref, gp_ref, gg_ref, cp_ref, cg_ref, temp_ref,
          out_ref, runmin_ref, acc_ref):
    b = pl.program_id(0)
    j = pl.program_id(1)

    @pl.when((b == 0) & (j == 0))
    def _init():
        gp = gp_ref[...]
        gg = gg_ref[...]
        pv = cp_ref[...]
        gv = cg_ref[...]
        c_pred = gp[:, 0:3]
        q = gp[:, 3:7]
        qn = jnp.sqrt(jnp.sum(q * q, axis=1, keepdims=True))
        q = q / jnp.maximum(qn, 1e-8)
        tr = gg[:, 0:1] + gg[:, 5:6] + gg[:, 10:11]
        w = 0.5 * jnp.sqrt(jnp.maximum(1.0 + tr, 1e-8))
        qx = (gg[:, 9:10] - gg[:, 6:7]) / (4.0 * w)
        qy = (gg[:, 2:3] - gg[:, 8:9]) / (4.0 * w)
        qz = (gg[:, 4:5] - gg[:, 1:2]) / (4.0 * w)
        dx = c_pred[:, 0:1] - gg[:, 3:4] + 1e-6
        dy = c_pred[:, 1:2] - gg[:, 7:8] + 1e-6
        dz = c_pred[:, 2:3] - gg[:, 11:12] + 1e-6
        trans = jnp.sqrt(dx * dx + dy * dy + dz * dz)
        dotq = q[:, 0:1] * w + q[:, 1:2] * qx + q[:, 2:3] * qy + q[:, 3:4] * qz
        bce = -(gv * jnp.log(pv) + (1.0 - gv) * jnp.log(1.0 - pv))
        acc_ref[0] = 0.0
        acc_ref[1] = 0.0
        acc_ref[2] = 0.0
        acc_ref[3] = jnp.sum(trans)
        acc_ref[4] = jnp.sum(1.0 - dotq)
        acc_ref[5] = jnp.sum(bce)

    ss = sample_ref[0]
    xxt = xyzt_ref[0]
    n1 = jnp.sum(ss * ss, axis=1, keepdims=True)
    n2 = jnp.sum(xxt * xxt, axis=0, keepdims=True)
    dot = jax.lax.dot_general(
        ss, xxt, (((1,), (0,)), ((), ())),
        preferred_element_type=jnp.float32,
        precision=jax.lax.Precision.HIGHEST)
    d2 = jnp.maximum(n1 + n2 - 2.0 * dot, 0.0)

    rowmin = jnp.min(d2, axis=1, keepdims=True)

    @pl.when(j == 0)
    def _first():
        runmin_ref[...] = rowmin

    @pl.when(j > 0)
    def _rest():
        runmin_ref[...] = jnp.minimum(runmin_ref[...], rowmin)

    colmin = jnp.min(d2, axis=0)
    acc_ref[2] = acc_ref[2] + jnp.sum(colmin)

    @pl.when(j == NSTEPS - 1)
    def _batch_done():
        rm = runmin_ref[...]
        acc_ref[0] = acc_ref[0] + jnp.sum(rm)
        acc_ref[1] = acc_ref[1] + jnp.max(rm)

    @pl.when((b == B - 1) & (j == NSTEPS - 1))
    def _fin():
        sample_loss = (acc_ref[0] / (B * M) + acc_ref[1] / B
                       + GAMMA * acc_ref[2] / (B * N))
        t = temp_ref[0, 0]
        reg = acc_ref[3] / (B * M) + ALPHA * acc_ref[4] / (B * M)
        cls = acc_ref[5] / (B * M)
        out_ref[0, 0] = THETA * sample_loss + t * t + BETA * reg + cls


@jax.jit
def _impl(xyz, sample_xyz, temp, grasp_pred, grasp_gt, class_pred, class_gt):
    xyzt = jnp.transpose(xyz, (0, 2, 1))
    gp = grasp_pred.reshape(B * M, 7)
    gg = grasp_gt.reshape(B * M, 16)
    cp = class_pred.reshape(B * M, 1)
    cg = class_gt.reshape(B * M, 1)
    tt = temp.reshape(1, 1)
    out = pl.pallas_call(
        _body,
        grid=(B, NSTEPS),
        in_specs=[
            pl.BlockSpec((1, M, 3), lambda b, j: (b, 0, 0)),
            pl.BlockSpec((1, 3, NB), lambda b, j: (b, 0, j)),
            pl.BlockSpec((B * M, 7), lambda b, j: (0, 0)),
            pl.BlockSpec((B * M, 16), lambda b, j: (0, 0)),
            pl.BlockSpec((B * M, 1), lambda b, j: (0, 0)),
            pl.BlockSpec((B * M, 1), lambda b, j: (0, 0)),
            pl.BlockSpec((1, 1), lambda b, j: (0, 0), memory_space=pltpu.SMEM),
        ],
        out_specs=pl.BlockSpec((1, 1), lambda b, j: (0, 0),
                               memory_space=pltpu.SMEM),
        out_shape=jax.ShapeDtypeStruct((1, 1), jnp.float32),
        scratch_shapes=[pltpu.VMEM((M, 1), jnp.float32),
                        pltpu.SMEM((8,), jnp.float32)],
    )(sample_xyz, xyzt, gp, gg, cp, cg, tt)
    return out[0, 0]


def kernel(xyz, sample_xyz, temp, grasp_pred, grasp_gt, class_pred, class_gt):
    return _impl(xyz, sample_xyz, temp, grasp_pred, grasp_gt,
                 class_pred, class_gt)


# speedup vs baseline: 2.7785x; 2.7785x over previous
import jax
import jax.numpy as jnp
from jax.experimental import pallas as pl
from jax.experimental.pallas import tpu as pltpu

GAMMA = 1.0
ALPHA = 0.5
BETA = 10.0
THETA = 1.0

B, N, M = 4, 16384, 1024
NB = 2048
NSTEPS = N // NB


def _body(sample_ref, xyzt_ref, gp_ref, gg_ref, cp_ref, cg_ref, temp_ref,
          out_ref, runmin_ref, acc_ref):
    b = pl.program_id(0)
    j = pl.program_id(1)

    @pl.when((b == 0) & (j == 0))
    def _init():
        gp = gp_ref[...]
        gg = gg_ref[...]
        pv = cp_ref[...]
        gv = cg_ref[...]
        c_pred = gp[:, 0:3]
        q = gp[:, 3:7]
        qn = jnp.sqrt(jnp.sum(q * q, axis=1, keepdims=True))
        q = q / jnp.maximum(qn, 1e-8)
        tr = gg[:, 0:1] + gg[:, 5:6] + gg[:, 10:11]
        w = 0.5 * jnp.sqrt(jnp.maximum(1.0 + tr, 1e-8))
        qx = (gg[:, 9:10] - gg[:, 6:7]) / (4.0 * w)
        qy = (gg[:, 2:3] - gg[:, 8:9]) / (4.0 * w)
        qz = (gg[:, 4:5] - gg[:, 1:2]) / (4.0 * w)
        dx = c_pred[:, 0:1] - gg[:, 3:4] + 1e-6
        dy = c_pred[:, 1:2] - gg[:, 7:8] + 1e-6
        dz = c_pred[:, 2:3] - gg[:, 11:12] + 1e-6
        trans = jnp.sqrt(dx * dx + dy * dy + dz * dz)
        dotq = q[:, 0:1] * w + q[:, 1:2] * qx + q[:, 2:3] * qy + q[:, 3:4] * qz
        bce = -(gv * jnp.log(pv) + (1.0 - gv) * jnp.log(1.0 - pv))
        acc_ref[0] = 0.0
        acc_ref[1] = 0.0
        acc_ref[2] = 0.0
        acc_ref[3] = jnp.sum(trans)
        acc_ref[4] = jnp.sum(1.0 - dotq)
        acc_ref[5] = jnp.sum(bce)

    ss = sample_ref[0]
    xxt = xyzt_ref[0]
    n1 = jnp.sum(ss * ss, axis=1, keepdims=True)
    n2 = jnp.sum(xxt * xxt, axis=0, keepdims=True)
    sa = jnp.concatenate([-2.0 * ss, n1, jnp.ones((M, 1), jnp.float32)],
                         axis=1)
    xa = jnp.concatenate([xxt, jnp.ones((1, NB), jnp.float32), n2],
                         axis=0)
    d2 = jax.lax.dot_general(
        sa, xa, (((1,), (0,)), ((), ())),
        preferred_element_type=jnp.float32,
        precision=jax.lax.Precision.DEFAULT)

    rowmin = jnp.maximum(jnp.min(d2, axis=1, keepdims=True), 0.0)

    @pl.when(j == 0)
    def _first():
        runmin_ref[...] = rowmin

    @pl.when(j > 0)
    def _rest():
        runmin_ref[...] = jnp.minimum(runmin_ref[...], rowmin)

    colmin = jnp.maximum(jnp.min(d2, axis=0), 0.0)
    acc_ref[2] = acc_ref[2] + jnp.sum(colmin)

    @pl.when(j == NSTEPS - 1)
    def _batch_done():
        rm = runmin_ref[...]
        acc_ref[0] = acc_ref[0] + jnp.sum(rm)
        acc_ref[1] = acc_ref[1] + jnp.max(rm)

    @pl.when((b == B - 1) & (j == NSTEPS - 1))
    def _fin():
        sample_loss = (acc_ref[0] / (B * M) + acc_ref[1] / B
                       + GAMMA * acc_ref[2] / (B * N))
        t = temp_ref[0, 0]
        reg = acc_ref[3] / (B * M) + ALPHA * acc_ref[4] / (B * M)
        cls = acc_ref[5] / (B * M)
        out_ref[0, 0] = THETA * sample_loss + t * t + BETA * reg + cls


@jax.jit
def _impl(xyz, sample_xyz, temp, grasp_pred, grasp_gt, class_pred, class_gt):
    xyzt = jnp.transpose(xyz, (0, 2, 1))
    gp = grasp_pred.reshape(B * M, 7)
    gg = grasp_gt.reshape(B * M, 16)
    cp = class_pred.reshape(B * M, 1)
    cg = class_gt.reshape(B * M, 1)
    tt = temp.reshape(1, 1)
    out = pl.pallas_call(
        _body,
        grid=(B, NSTEPS),
        in_specs=[
            pl.BlockSpec((1, M, 3), lambda b, j: (b, 0, 0)),
            pl.BlockSpec((1, 3, NB), lambda b, j: (b, 0, j)),
            pl.BlockSpec((B * M, 7), lambda b, j: (0, 0)),
            pl.BlockSpec((B * M, 16), lambda b, j: (0, 0)),
            pl.BlockSpec((B * M, 1), lambda b, j: (0, 0)),
            pl.BlockSpec((B * M, 1), lambda b, j: (0, 0)),
            pl.BlockSpec((1, 1), lambda b, j: (0, 0), memory_space=pltpu.SMEM),
        ],
        out_specs=pl.BlockSpec((1, 1), lambda b, j: (0, 0),
                               memory_space=pltpu.SMEM),
        out_shape=jax.ShapeDtypeStruct((1, 1), jnp.float32),
        scratch_shapes=[pltpu.VMEM((M, 1), jnp.float32),
                        pltpu.SMEM((8,), jnp.float32)],
    )(sample_xyz, xyzt, gp, gg, cp, cg, tt)
    return out[0, 0]


def kernel(xyz, sample_xyz, temp, grasp_pred, grasp_gt, class_pred, class_gt):
    return _impl(xyz, sample_xyz, temp, grasp_pred, grasp_gt,
                 class_pred, class_gt)


# speedup vs baseline: 3.9727x; 1.4298x over previous
import jax
import jax.numpy as jnp
from jax.experimental import pallas as pl
from jax.experimental.pallas import tpu as pltpu

GAMMA = 1.0
ALPHA = 0.5
BETA = 10.0
THETA = 1.0

B, N, M = 4, 16384, 1024
NB = 4096
NSTEPS = N // NB


def _body(sample_ref, xyzt_ref, gp_ref, gg_ref, cp_ref, cg_ref, temp_ref,
          out_ref, runmin_ref, acc_ref):
    b = pl.program_id(0)
    j = pl.program_id(1)

    @pl.when((b == 0) & (j == 0))
    def _init():
        gp = gp_ref[...]
        gg = gg_ref[...]
        pv = cp_ref[...]
        gv = cg_ref[...]
        q0, q1, q2, q3 = gp[3], gp[4], gp[5], gp[6]
        qn = jnp.sqrt(q0 * q0 + q1 * q1 + q2 * q2 + q3 * q3)
        qinv = 1.0 / jnp.maximum(qn, 1e-8)
        tr = gg[0] + gg[5] + gg[10]
        w = 0.5 * jnp.sqrt(jnp.maximum(1.0 + tr, 1e-8))
        inv4w = 0.25 / w
        qx = (gg[9] - gg[6]) * inv4w
        qy = (gg[2] - gg[8]) * inv4w
        qz = (gg[4] - gg[1]) * inv4w
        dx = gp[0] - gg[3] + 1e-6
        dy = gp[1] - gg[7] + 1e-6
        dz = gp[2] - gg[11] + 1e-6
        trans = jnp.sqrt(dx * dx + dy * dy + dz * dz)
        dotq = (q0 * w + q1 * qx + q2 * qy + q3 * qz) * qinv
        bce = -(gv * jnp.log(pv) + (1.0 - gv) * jnp.log(1.0 - pv))
        acc_ref[0] = 0.0
        acc_ref[1] = 0.0
        acc_ref[2] = 0.0
        acc_ref[3] = jnp.sum(trans)
        acc_ref[4] = jnp.sum(1.0 - dotq)
        acc_ref[5] = jnp.sum(bce)

    ss = sample_ref[0]
    xxt = xyzt_ref[0]
    n1 = jnp.sum(ss * ss, axis=1, keepdims=True)
    n2 = jnp.sum(xxt * xxt, axis=0, keepdims=True)
    sa = jnp.concatenate([-2.0 * ss, n1, jnp.ones((M, 1), jnp.float32)],
                         axis=1)
    xa = jnp.concatenate([xxt, jnp.ones((1, NB), jnp.float32), n2],
                         axis=0)
    d2 = jax.lax.dot_general(
        sa, xa, (((1,), (0,)), ((), ())),
        preferred_element_type=jnp.float32,
        precision=jax.lax.Precision.DEFAULT)

    rowmin = jnp.maximum(jnp.min(d2, axis=1, keepdims=True), 0.0)

    @pl.when(j == 0)
    def _first():
        runmin_ref[...] = rowmin

    @pl.when(j > 0)
    def _rest():
        runmin_ref[...] = jnp.minimum(runmin_ref[...], rowmin)

    colmin = jnp.maximum(jnp.min(d2, axis=0), 0.0)
    acc_ref[2] = acc_ref[2] + jnp.sum(colmin)

    @pl.when(j == NSTEPS - 1)
    def _batch_done():
        rm = runmin_ref[...]
        acc_ref[0] = acc_ref[0] + jnp.sum(rm)
        acc_ref[1] = acc_ref[1] + jnp.max(rm)

    @pl.when((b == B - 1) & (j == NSTEPS - 1))
    def _fin():
        sample_loss = (acc_ref[0] / (B * M) + acc_ref[1] / B
                       + GAMMA * acc_ref[2] / (B * N))
        t = temp_ref[0, 0]
        reg = acc_ref[3] / (B * M) + ALPHA * acc_ref[4] / (B * M)
        cls = acc_ref[5] / (B * M)
        out_ref[0, 0] = THETA * sample_loss + t * t + BETA * reg + cls


@jax.jit
def _impl(xyz, sample_xyz, temp, grasp_pred, grasp_gt, class_pred, class_gt):
    xyzt = jnp.transpose(xyz, (0, 2, 1))
    gp = grasp_pred.reshape(B * M, 7).T.reshape(7, 32, 128)
    gg = grasp_gt.reshape(B * M, 16).T.reshape(16, 32, 128)
    cp = class_pred.reshape(32, 128)
    cg = class_gt.reshape(32, 128)
    tt = temp.reshape(1, 1)
    out = pl.pallas_call(
        _body,
        grid=(B, NSTEPS),
        in_specs=[
            pl.BlockSpec((1, M, 3), lambda b, j: (b, 0, 0)),
            pl.BlockSpec((1, 3, NB), lambda b, j: (b, 0, j)),
            pl.BlockSpec((7, 32, 128), lambda b, j: (0, 0, 0)),
            pl.BlockSpec((16, 32, 128), lambda b, j: (0, 0, 0)),
            pl.BlockSpec((32, 128), lambda b, j: (0, 0)),
            pl.BlockSpec((32, 128), lambda b, j: (0, 0)),
            pl.BlockSpec((1, 1), lambda b, j: (0, 0), memory_space=pltpu.SMEM),
        ],
        out_specs=pl.BlockSpec((1, 1), lambda b, j: (0, 0),
                               memory_space=pltpu.SMEM),
        out_shape=jax.ShapeDtypeStruct((1, 1), jnp.float32),
        scratch_shapes=[pltpu.VMEM((M, 1), jnp.float32),
                        pltpu.SMEM((8,), jnp.float32)],
    )(sample_xyz, xyzt, gp, gg, cp, cg, tt)
    return out[0, 0]


def kernel(xyz, sample_xyz, temp, grasp_pred, grasp_gt, class_pred, class_gt):
    return _impl(xyz, sample_xyz, temp, grasp_pred, grasp_gt,
                 class_pred, class_gt)


# speedup vs baseline: 4.2392x; 1.0671x over previous
import jax
import jax.numpy as jnp
from jax.experimental import pallas as pl
from jax.experimental.pallas import tpu as pltpu

GAMMA = 1.0
ALPHA = 0.5
BETA = 10.0
THETA = 1.0

B, N, M = 4, 16384, 1024
NB = 8192
NSTEPS = N // NB


def _body(sample_ref, xyzt_ref, gp_ref, gg_ref, cp_ref, cg_ref, temp_ref,
          out_ref, runmin_ref, acc_ref):
    b = pl.program_id(0)
    j = pl.program_id(1)

    @pl.when((b == 0) & (j == 0))
    def _init():
        gp = gp_ref[...]
        gg = gg_ref[...]
        pv = cp_ref[...]
        gv = cg_ref[...]
        q0, q1, q2, q3 = gp[3], gp[4], gp[5], gp[6]
        qn = jnp.sqrt(q0 * q0 + q1 * q1 + q2 * q2 + q3 * q3)
        qinv = 1.0 / jnp.maximum(qn, 1e-8)
        tr = gg[0] + gg[5] + gg[10]
        w = 0.5 * jnp.sqrt(jnp.maximum(1.0 + tr, 1e-8))
        inv4w = 0.25 / w
        qx = (gg[9] - gg[6]) * inv4w
        qy = (gg[2] - gg[8]) * inv4w
        qz = (gg[4] - gg[1]) * inv4w
        dx = gp[0] - gg[3] + 1e-6
        dy = gp[1] - gg[7] + 1e-6
        dz = gp[2] - gg[11] + 1e-6
        trans = jnp.sqrt(dx * dx + dy * dy + dz * dz)
        dotq = (q0 * w + q1 * qx + q2 * qy + q3 * qz) * qinv
        bce = -(gv * jnp.log(pv) + (1.0 - gv) * jnp.log(1.0 - pv))
        acc_ref[0] = 0.0
        acc_ref[1] = 0.0
        acc_ref[2] = 0.0
        acc_ref[3] = jnp.sum(trans)
        acc_ref[4] = jnp.sum(1.0 - dotq)
        acc_ref[5] = jnp.sum(bce)

    ss = sample_ref[0]
    xxt = xyzt_ref[0]
    n1 = jnp.sum(ss * ss, axis=1, keepdims=True)
    n2 = jnp.sum(xxt * xxt, axis=0, keepdims=True)
    sa = jnp.concatenate([-2.0 * ss, n1, jnp.ones((M, 1), jnp.float32)],
                         axis=1)
    xa = jnp.concatenate([xxt, jnp.ones((1, NB), jnp.float32), n2],
                         axis=0)
    d2 = jax.lax.dot_general(
        sa, xa, (((1,), (0,)), ((), ())),
        preferred_element_type=jnp.float32,
        precision=jax.lax.Precision.DEFAULT)

    rowmin = jnp.maximum(jnp.min(d2, axis=1, keepdims=True), 0.0)

    @pl.when(j == 0)
    def _first():
        runmin_ref[...] = rowmin

    @pl.when(j > 0)
    def _rest():
        runmin_ref[...] = jnp.minimum(runmin_ref[...], rowmin)

    colmin = jnp.maximum(jnp.min(d2, axis=0), 0.0)
    acc_ref[2] = acc_ref[2] + jnp.sum(colmin)

    @pl.when(j == NSTEPS - 1)
    def _batch_done():
        rm = runmin_ref[...]
        acc_ref[0] = acc_ref[0] + jnp.sum(rm)
        acc_ref[1] = acc_ref[1] + jnp.max(rm)

    @pl.when((b == B - 1) & (j == NSTEPS - 1))
    def _fin():
        sample_loss = (acc_ref[0] / (B * M) + acc_ref[1] / B
                       + GAMMA * acc_ref[2] / (B * N))
        t = temp_ref[0, 0]
        reg = acc_ref[3] / (B * M) + ALPHA * acc_ref[4] / (B * M)
        cls = acc_ref[5] / (B * M)
        out_ref[0, 0] = THETA * sample_loss + t * t + BETA * reg + cls


@jax.jit
def _impl(xyz, sample_xyz, temp, grasp_pred, grasp_gt, class_pred, class_gt):
    xyzt = jnp.transpose(xyz, (0, 2, 1))
    gp = grasp_pred.reshape(B * M, 7).T.reshape(7, 32, 128)
    gg = grasp_gt.reshape(B * M, 16).T.reshape(16, 32, 128)
    cp = class_pred.reshape(32, 128)
    cg = class_gt.reshape(32, 128)
    tt = temp.reshape(1, 1)
    out = pl.pallas_call(
        _body,
        grid=(B, NSTEPS),
        in_specs=[
            pl.BlockSpec((1, M, 3), lambda b, j: (b, 0, 0)),
            pl.BlockSpec((1, 3, NB), lambda b, j: (b, 0, j)),
            pl.BlockSpec((7, 32, 128), lambda b, j: (0, 0, 0)),
            pl.BlockSpec((16, 32, 128), lambda b, j: (0, 0, 0)),
            pl.BlockSpec((32, 128), lambda b, j: (0, 0)),
            pl.BlockSpec((32, 128), lambda b, j: (0, 0)),
            pl.BlockSpec((1, 1), lambda b, j: (0, 0), memory_space=pltpu.SMEM),
        ],
        out_specs=pl.BlockSpec((1, 1), lambda b, j: (0, 0),
                               memory_space=pltpu.SMEM),
        out_shape=jax.ShapeDtypeStruct((1, 1), jnp.float32),
        scratch_shapes=[pltpu.VMEM((M, 1), jnp.float32),
                        pltpu.SMEM((8,), jnp.float32)],
    )(sample_xyz, xyzt, gp, gg, cp, cg, tt)
    return out[0, 0]


def kernel(xyz, sample_xyz, temp, grasp_pred, grasp_gt, class_pred, class_gt):
    return _impl(xyz, sample_xyz, temp, grasp_pred, grasp_gt,
                 class_pred, class_gt)


# speedup vs baseline: 5.2304x; 1.2338x over previous
import jax
import jax.numpy as jnp
from jax.experimental import pallas as pl
from jax.experimental.pallas import tpu as pltpu

GAMMA = 1.0
ALPHA = 0.5
BETA = 10.0
THETA = 1.0

B, N, M = 4, 16384, 1024
NB = 8192
CH = 2048
NSTEPS = N // NB


def _body(sample_ref, xyzt_ref, gp_ref, gg_ref, cp_ref, cg_ref, temp_ref,
          out_ref, runmin_ref, acc_ref):
    b = pl.program_id(0)
    j = pl.program_id(1)

    @pl.when((b == 0) & (j == 0))
    def _init():
        gp = gp_ref[...]
        gg = gg_ref[...]
        pv = cp_ref[...]
        gv = cg_ref[...]
        q0, q1, q2, q3 = gp[3], gp[4], gp[5], gp[6]
        qn = jnp.sqrt(q0 * q0 + q1 * q1 + q2 * q2 + q3 * q3)
        qinv = 1.0 / jnp.maximum(qn, 1e-8)
        tr = gg[0] + gg[5] + gg[10]
        w = 0.5 * jnp.sqrt(jnp.maximum(1.0 + tr, 1e-8))
        inv4w = 0.25 / w
        qx = (gg[9] - gg[6]) * inv4w
        qy = (gg[2] - gg[8]) * inv4w
        qz = (gg[4] - gg[1]) * inv4w
        dx = gp[0] - gg[3] + 1e-6
        dy = gp[1] - gg[7] + 1e-6
        dz = gp[2] - gg[11] + 1e-6
        trans = jnp.sqrt(dx * dx + dy * dy + dz * dz)
        dotq = (q0 * w + q1 * qx + q2 * qy + q3 * qz) * qinv
        bce = -(gv * jnp.log(pv) + (1.0 - gv) * jnp.log(1.0 - pv))
        acc_ref[0] = 0.0
        acc_ref[1] = 0.0
        acc_ref[2] = 0.0
        acc_ref[3] = jnp.sum(trans)
        acc_ref[4] = jnp.sum(1.0 - dotq)
        acc_ref[5] = jnp.sum(bce)

    ss = sample_ref[0]
    xxt = xyzt_ref[0]
    n1 = jnp.sum(ss * ss, axis=1, keepdims=True)
    n2 = jnp.sum(xxt * xxt, axis=0, keepdims=True)
    sa = jnp.concatenate([-2.0 * ss, n1, jnp.ones((M, 1), jnp.float32)],
                         axis=1)
    xa = jnp.concatenate([xxt, jnp.ones((1, NB), jnp.float32), n2],
                         axis=0)

    rowmin = None
    ossum = jnp.float32(0.0)
    for c in range(NB // CH):
        d2 = jax.lax.dot_general(
            sa, xa[:, c * CH:(c + 1) * CH], (((1,), (0,)), ((), ())),
            preferred_element_type=jnp.float32,
            precision=jax.lax.Precision.DEFAULT)
        rm = jnp.min(d2, axis=1, keepdims=True)
        rowmin = rm if rowmin is None else jnp.minimum(rowmin, rm)
        cm = jnp.min(d2, axis=0)
        ossum = ossum + jnp.sum(jnp.maximum(cm, 0.0))

    rowmin = jnp.maximum(rowmin, 0.0)

    @pl.when(j == 0)
    def _first():
        runmin_ref[...] = rowmin

    @pl.when(j > 0)
    def _rest():
        runmin_ref[...] = jnp.minimum(runmin_ref[...], rowmin)

    acc_ref[2] = acc_ref[2] + ossum

    @pl.when(j == NSTEPS - 1)
    def _batch_done():
        rm = runmin_ref[...]
        acc_ref[0] = acc_ref[0] + jnp.sum(rm)
        acc_ref[1] = acc_ref[1] + jnp.max(rm)

    @pl.when((b == B - 1) & (j == NSTEPS - 1))
    def _fin():
        sample_loss = (acc_ref[0] / (B * M) + acc_ref[1] / B
                       + GAMMA * acc_ref[2] / (B * N))
        t = temp_ref[0, 0]
        reg = acc_ref[3] / (B * M) + ALPHA * acc_ref[4] / (B * M)
        cls = acc_ref[5] / (B * M)
        out_ref[0, 0] = THETA * sample_loss + t * t + BETA * reg + cls


@jax.jit
def _impl(xyz, sample_xyz, temp, grasp_pred, grasp_gt, class_pred, class_gt):
    xyzt = jnp.transpose(xyz, (0, 2, 1))
    gp = grasp_pred.reshape(B * M, 7).T.reshape(7, 32, 128)
    gg = grasp_gt.reshape(B * M, 16).T.reshape(16, 32, 128)
    cp = class_pred.reshape(32, 128)
    cg = class_gt.reshape(32, 128)
    tt = temp.reshape(1, 1)
    out = pl.pallas_call(
        _body,
        grid=(B, NSTEPS),
        in_specs=[
            pl.BlockSpec((1, M, 3), lambda b, j: (b, 0, 0)),
            pl.BlockSpec((1, 3, NB), lambda b, j: (b, 0, j)),
            pl.BlockSpec((7, 32, 128), lambda b, j: (0, 0, 0)),
            pl.BlockSpec((16, 32, 128), lambda b, j: (0, 0, 0)),
            pl.BlockSpec((32, 128), lambda b, j: (0, 0)),
            pl.BlockSpec((32, 128), lambda b, j: (0, 0)),
            pl.BlockSpec((1, 1), lambda b, j: (0, 0), memory_space=pltpu.SMEM),
        ],
        out_specs=pl.BlockSpec((1, 1), lambda b, j: (0, 0),
                               memory_space=pltpu.SMEM),
        out_shape=jax.ShapeDtypeStruct((1, 1), jnp.float32),
        scratch_shapes=[pltpu.VMEM((M, 1), jnp.float32),
                        pltpu.SMEM((8,), jnp.float32)],
    )(sample_xyz, xyzt, gp, gg, cp, cg, tt)
    return out[0, 0]


def kernel(xyz, sample_xyz, temp, grasp_pred, grasp_gt, class_pred, class_gt):
    return _impl(xyz, sample_xyz, temp, grasp_pred, grasp_gt,
                 class_pred, class_gt)


# speedup vs baseline: 5.4811x; 1.0479x over previous
import jax
import jax.numpy as jnp
from jax.experimental import pallas as pl
from jax.experimental.pallas import tpu as pltpu

GAMMA = 1.0
ALPHA = 0.5
BETA = 10.0
THETA = 1.0

B, N, M = 4, 16384, 1024
CH = 2048


def _body(sample_ref, xyzt_ref, gp_ref, gg_ref, cp_ref, cg_ref, temp_ref,
          out_ref, acc_ref):
    b = pl.program_id(0)

    @pl.when(b == 0)
    def _init():
        gp = gp_ref[...]
        gg = gg_ref[...]
        pv = cp_ref[...]
        gv = cg_ref[...]
        q0, q1, q2, q3 = gp[3], gp[4], gp[5], gp[6]
        qn = jnp.sqrt(q0 * q0 + q1 * q1 + q2 * q2 + q3 * q3)
        qinv = 1.0 / jnp.maximum(qn, 1e-8)
        tr = gg[0] + gg[5] + gg[10]
        w = 0.5 * jnp.sqrt(jnp.maximum(1.0 + tr, 1e-8))
        inv4w = 0.25 / w
        qx = (gg[9] - gg[6]) * inv4w
        qy = (gg[2] - gg[8]) * inv4w
        qz = (gg[4] - gg[1]) * inv4w
        dx = gp[0] - gg[3] + 1e-6
        dy = gp[1] - gg[7] + 1e-6
        dz = gp[2] - gg[11] + 1e-6
        trans = jnp.sqrt(dx * dx + dy * dy + dz * dz)
        dotq = (q0 * w + q1 * qx + q2 * qy + q3 * qz) * qinv
        bce = -(gv * jnp.log(pv) + (1.0 - gv) * jnp.log(1.0 - pv))
        acc_ref[0] = 0.0
        acc_ref[1] = 0.0
        acc_ref[2] = 0.0
        acc_ref[3] = jnp.sum(trans)
        acc_ref[4] = jnp.sum(1.0 - dotq)
        acc_ref[5] = jnp.sum(bce)

    ss = sample_ref[0]
    xxt = xyzt_ref[0]
    n1 = jnp.sum(ss * ss, axis=1, keepdims=True)
    n2 = jnp.sum(xxt * xxt, axis=0, keepdims=True)
    sa = jnp.concatenate([-2.0 * ss, n1, jnp.ones((M, 1), jnp.float32)],
                         axis=1)
    xa = jnp.concatenate([xxt, jnp.ones((1, N), jnp.float32), n2],
                         axis=0)

    rowmin = None
    ossum = jnp.float32(0.0)
    for c in range(N // CH):
        d2 = jax.lax.dot_general(
            sa, xa[:, c * CH:(c + 1) * CH], (((1,), (0,)), ((), ())),
            preferred_element_type=jnp.float32,
            precision=jax.lax.Precision.DEFAULT)
        rm = jnp.min(d2, axis=1, keepdims=True)
        rowmin = rm if rowmin is None else jnp.minimum(rowmin, rm)
        cm = jnp.min(d2, axis=0)
        ossum = ossum + jnp.sum(jnp.maximum(cm, 0.0))

    rowmin = jnp.maximum(rowmin, 0.0)

    acc_ref[0] = acc_ref[0] + jnp.sum(rowmin)
    acc_ref[1] = acc_ref[1] + jnp.max(rowmin)
    acc_ref[2] = acc_ref[2] + ossum

    @pl.when(b == B - 1)
    def _fin():
        sample_loss = (acc_ref[0] / (B * M) + acc_ref[1] / B
                       + GAMMA * acc_ref[2] / (B * N))
        t = temp_ref[0, 0]
        reg = acc_ref[3] / (B * M) + ALPHA * acc_ref[4] / (B * M)
        cls = acc_ref[5] / (B * M)
        out_ref[0, 0] = THETA * sample_loss + t * t + BETA * reg + cls


@jax.jit
def _impl(xyz, sample_xyz, temp, grasp_pred, grasp_gt, class_pred, class_gt):
    xyzt = jnp.transpose(xyz, (0, 2, 1))
    gp = grasp_pred.reshape(B * M, 7).T.reshape(7, 32, 128)
    gg = grasp_gt.reshape(B * M, 16).T.reshape(16, 32, 128)
    cp = class_pred.reshape(32, 128)
    cg = class_gt.reshape(32, 128)
    tt = temp.reshape(1, 1)
    out = pl.pallas_call(
        _body,
        grid=(B,),
        in_specs=[
            pl.BlockSpec((1, M, 3), lambda b: (b, 0, 0)),
            pl.BlockSpec((1, 3, N), lambda b: (b, 0, 0)),
            pl.BlockSpec((7, 32, 128), lambda b: (0, 0, 0)),
            pl.BlockSpec((16, 32, 128), lambda b: (0, 0, 0)),
            pl.BlockSpec((32, 128), lambda b: (0, 0)),
            pl.BlockSpec((32, 128), lambda b: (0, 0)),
            pl.BlockSpec((1, 1), lambda b: (0, 0), memory_space=pltpu.SMEM),
        ],
        out_specs=pl.BlockSpec((1, 1), lambda b: (0, 0),
                               memory_space=pltpu.SMEM),
        out_shape=jax.ShapeDtypeStruct((1, 1), jnp.float32),
        scratch_shapes=[pltpu.SMEM((8,), jnp.float32)],
    )(sample_xyz, xyzt, gp, gg, cp, cg, tt)
    return out[0, 0]


def kernel(xyz, sample_xyz, temp, grasp_pred, grasp_gt, class_pred, class_gt):
    return _impl(xyz, sample_xyz, temp, grasp_pred, grasp_gt,
                 class_pred, class_gt)
